# Initial kernel scaffold; baseline (speedup 1.0000x reference)
import jax
import jax.numpy as jnp
from jax.experimental import pallas as pl


def kernel(x_enc, x_mark_enc, x_dec, x_mark_dec, params):
    raise NotImplementedError("write your pallas kernel here")


# speedup vs baseline: 2.3465x; 2.3465x over previous
import functools

import jax
import jax.numpy as jnp
from jax.experimental import pallas as pl

B, L, N, C = 4, 96, 170, 1
D, H, DFF, E, TOPK = 128, 8, 512, 4, 2
NLAYERS = 3
SLICE = 288
PRED = 96
EPS = 1e-5
DH = D // H

F32 = jnp.float32


def _dot(a, b):
    return jax.lax.dot_general(a, b, (((1,), (0,)), ((), ())),
                               preferred_element_type=F32)


def _dot_t(a, b):
    return jax.lax.dot_general(a, b, (((1,), (1,)), ((), ())),
                               preferred_element_type=F32)


def _ln(x, g, b):
    mu = jnp.mean(x, axis=-1, keepdims=True)
    xc = x - mu
    var = jnp.mean(xc * xc, axis=-1, keepdims=True)
    return xc * jax.lax.rsqrt(var + EPS) * g + b


def _top2_weights(logits):
    cols = [logits[:, e:e + 1] for e in range(E)]
    sels = []
    for e in range(E):
        rank = None
        for j in range(E):
            if j == e:
                continue
            gt = cols[j] > cols[e]
            if j < e:
                gt = gt | (cols[j] == cols[e])
            r = gt.astype(F32)
            rank = r if rank is None else rank + r
        sels.append(rank < 2.0)
    neg = jnp.float32(-1e30)
    m = None
    for e in range(E):
        v = jnp.where(sels[e], cols[e], neg)
        m = v if m is None else jnp.maximum(m, v)
    ws = []
    z = None
    for e in range(E):
        w = jnp.exp(cols[e] - m) * sels[e].astype(F32)
        ws.append(w)
        z = w if z is None else z + w
    inv = 1.0 / z
    return [w * inv for w in ws]


def _moe_dense(x, gate_w, gate_b, w1t, b1, w2t, b2):
    logits = _dot_t(x, gate_w) + gate_b
    ws = _top2_weights(logits)
    acc = None
    for e in range(E):
        h = jnp.maximum(_dot(x, w1t[e]) + b1[e:e + 1, :], 0.0)
        y = _dot(h, w2t[e]) + b2[e:e + 1, :]
        y = y * ws[e]
        acc = y if acc is None else acc + y
    return acc


def _spatial_kernel(xs_ref, spwt_ref, wvt_ref, bv_ref, wot_ref, bo_ref,
                    ln2g_ref, ln2b_ref, ln3g_ref, ln3b_ref,
                    gw_ref, gb_ref, w1t_ref, b1_ref, w2t_ref, b2_ref,
                    p6_ref, out_ref):
    se = _dot(xs_ref[...], spwt_ref[...])
    a = _dot(se, wvt_ref[...]) + bv_ref[...]
    a = _dot(a, wot_ref[...]) + bo_ref[...]
    x1 = _ln(se + a, ln2g_ref[...], ln2b_ref[...])
    f = _moe_dense(x1, gw_ref[...], gb_ref[...], w1t_ref, b1_ref[...],
                   w2t_ref, b2_ref[...])
    sp = _ln(x1 + f, ln3g_ref[...], ln3b_ref[...])
    out_ref[...] = _dot_t(sp, p6_ref[...])


def _spatial_call(xs, p, proj_w):
    ap = p['attn']
    mp = p['moe']
    w1t = jnp.stack([e['W1'].T for e in mp['experts']])
    b1 = jnp.stack([e['b1'] for e in mp['experts']])
    w2t = jnp.stack([e['W2'].T for e in mp['experts']])
    b2 = jnp.stack([e['b2'] for e in mp['experts']])
    args = (xs, p['spatial_W_T'], ap['Wv'].T, ap['bv'][None], ap['Wo'].T,
            ap['bo'][None], p['ln2_g'][None], p['ln2_b'][None],
            p['ln3_g'][None], p['ln3_b'][None],
            mp['gate_W'], mp['gate_b'][None], w1t, b1, w2t, b2,
            proj_w[:, 5 * D:6 * D])
    return pl.pallas_call(
        _spatial_kernel,
        out_shape=jax.ShapeDtypeStruct((B * N, D), F32),
    )(*args)


NT = 34


def _embed_kernel(trend_ref, season_ref, m6_ref, m2_ref, tod_idx_ref,
                  dow_idx_ref, tod_tab_ref, dow_tab_ref, p4_ref, p5_ref,
                  ut_ref, us_ref, u6_ref, u2_ref, cvec_ref,
                  adp_ref, s_ref, out_ref):
    ut = ut_ref[...].reshape(1, 1, D)
    us = us_ref[...].reshape(1, 1, D)
    u6 = u6_ref[...]
    u2 = u2_ref[...]
    h = trend_ref[0] * ut + season_ref[0] * us
    bl = m6_ref[0] * u6 + m2_ref[0] * u2 + cvec_ref[...]
    tp = _dot_t(tod_tab_ref[...], p4_ref[...])
    dp = _dot_t(dow_tab_ref[...], p5_ref[...])
    ti = tod_idx_ref[0]
    di = dow_idx_ref[0]
    oh_t = (jax.lax.broadcasted_iota(jnp.int32, (L, SLICE), 1) == ti
            ).astype(F32)
    oh_d = (jax.lax.broadcasted_iota(jnp.int32, (L, 7), 1) == di
            ).astype(F32)
    bl = bl + _dot(oh_t, tp) + _dot(oh_d, dp)
    h = h + bl[None, :, :]
    h = h + adp_ref[...]
    h = h + jnp.broadcast_to(s_ref[0], (NT, L, D))
    out_ref[0] = h


def _embed_call(trend, season, m6, m2, tod_idx, dow_idx, params, folded,
                adaptive_p, s_proj):
    proj_w = params['proj_W']
    grid = (B, N // NT)
    bs = [
        pl.BlockSpec((1, NT, L, 1), lambda b, j: (b, j, 0, 0)),
        pl.BlockSpec((1, NT, L, 1), lambda b, j: (b, j, 0, 0)),
        pl.BlockSpec((1, L, 1), lambda b, j: (b, 0, 0)),
        pl.BlockSpec((1, L, 1), lambda b, j: (b, 0, 0)),
        pl.BlockSpec((1, L, 1), lambda b, j: (b, 0, 0)),
        pl.BlockSpec((1, L, 1), lambda b, j: (b, 0, 0)),
        pl.BlockSpec((SLICE, D), lambda b, j: (0, 0)),
        pl.BlockSpec((7, D), lambda b, j: (0, 0)),
        pl.BlockSpec((D, D), lambda b, j: (0, 0)),
        pl.BlockSpec((D, D), lambda b, j: (0, 0)),
        pl.BlockSpec((1, D), lambda b, j: (0, 0)),
        pl.BlockSpec((1, D), lambda b, j: (0, 0)),
        pl.BlockSpec((1, D), lambda b, j: (0, 0)),
        pl.BlockSpec((1, D), lambda b, j: (0, 0)),
        pl.BlockSpec((1, D), lambda b, j: (0, 0)),
        pl.BlockSpec((NT, L, D), lambda b, j: (j, 0, 0)),
        pl.BlockSpec((1, NT, 1, D), lambda b, j: (b, j, 0, 0)),
    ]
    ut, us, u6, u2, cvec = folded
    return pl.pallas_call(
        _embed_kernel,
        grid=grid,
        in_specs=bs,
        out_specs=pl.BlockSpec((1, NT, L, D), lambda b, j: (b, j, 0, 0)),
        out_shape=jax.ShapeDtypeStruct((B, N, L, D), F32),
    )(trend[..., None], season[..., None], m6, m2, tod_idx, dow_idx,
      params['tod_table'], params['dow_table'],
      proj_w[:, 3 * D:4 * D], proj_w[:, 4 * D:5 * D],
      ut, us, u6, u2, cvec, adaptive_p, s_proj)


G = 20


def _block_kernel(x_ref, wqt_ref, bq_ref, wkt_ref, bk_ref, wvt_ref, bv_ref,
                  wot_ref, bo_ref, ln2g_ref, ln2b_ref, ln3g_ref, ln3b_ref,
                  gw_ref, gb_ref, w1t_ref, b1_ref, w2t_ref, b2_ref, out_ref):
    x3 = x_ref[...]
    x = x3.reshape(G * L, D)
    q = _dot(x, wqt_ref[...]) + bq_ref[...]
    k = _dot(x, wkt_ref[...]) + bk_ref[...]
    v = _dot(x, wvt_ref[...]) + bv_ref[...]
    scale = 1.0 / (DH ** 0.5)
    heads = []
    for h in range(H):
        qh = (q[:, h * DH:(h + 1) * DH] * scale).reshape(G, L, DH)
        kh = k[:, h * DH:(h + 1) * DH].reshape(G, L, DH)
        vh = v[:, h * DH:(h + 1) * DH].reshape(G, L, DH)
        s = jax.lax.dot_general(qh, kh, (((2,), (2,)), ((0,), (0,))),
                                preferred_element_type=F32)
        s = s - jnp.max(s, axis=-1, keepdims=True)
        p = jnp.exp(s)
        p = p / jnp.sum(p, axis=-1, keepdims=True)
        o = jax.lax.dot_general(p, vh, (((2,), (1,)), ((0,), (0,))),
                                preferred_element_type=F32)
        heads.append(o.reshape(G * L, DH))
    a = jnp.concatenate(heads, axis=1)
    a = _dot(a, wot_ref[...]) + bo_ref[...]
    x1 = _ln(x + a, ln2g_ref[...], ln2b_ref[...])
    f = _moe_dense(x1, gw_ref[...], gb_ref[...], w1t_ref, b1_ref[...],
                   w2t_ref, b2_ref[...])
    x2 = _ln(x1 + f, ln3g_ref[...], ln3b_ref[...])
    out_ref[...] = x2.reshape(G, L, D)


def _block_call(h, p):
    ap = p['attn']
    mp = p['moe']
    w1t = jnp.stack([e['W1'].T for e in mp['experts']])
    b1 = jnp.stack([e['b1'] for e in mp['experts']])
    w2t = jnp.stack([e['W2'].T for e in mp['experts']])
    b2 = jnp.stack([e['b2'] for e in mp['experts']])
    full = lambda shape: pl.BlockSpec(shape, lambda i: tuple(0 for _ in shape))
    bs = [pl.BlockSpec((G, L, D), lambda i: (i, 0, 0)),
          full((D, D)), full((1, D)), full((D, D)), full((1, D)),
          full((D, D)), full((1, D)), full((D, D)), full((1, D)),
          full((1, D)), full((1, D)), full((1, D)), full((1, D)),
          full((E, D)), full((1, E)), full((E, D, DFF)), full((E, DFF)),
          full((E, DFF, D)), full((E, D))]
    return pl.pallas_call(
        _block_kernel,
        grid=(B * N // G,),
        in_specs=bs,
        out_specs=pl.BlockSpec((G, L, D), lambda i: (i, 0, 0)),
        out_shape=jax.ShapeDtypeStruct((B * N, L, D), F32),
    )(h, ap['Wq'].T, ap['bq'][None], ap['Wk'].T, ap['bk'][None],
      ap['Wv'].T, ap['bv'][None], ap['Wo'].T, ap['bo'][None],
      p['ln2_g'][None], p['ln2_b'][None], p['ln3_g'][None], p['ln3_b'][None],
      mp['gate_W'], mp['gate_b'][None], w1t, b1, w2t, b2)


def _mix_kernel(h_ref, wt_ref, b_ref, st_ref, mu_ref, out_ref):
    y = _dot(h_ref[0], wt_ref[...]) + b_ref[...]
    out_ref[0] = y * st_ref[0, 0, 0] + mu_ref[0, 0, 0]


def _mix_call(hflat, mix_w, mix_b, stdev, mean):
    bs = [
        pl.BlockSpec((1, N, L * D), lambda b: (b, 0, 0)),
        pl.BlockSpec((L * D, PRED), lambda b: (0, 0)),
        pl.BlockSpec((1, PRED), lambda b: (0, 0)),
        pl.BlockSpec((1, 1, 1), lambda b: (b, 0, 0)),
        pl.BlockSpec((1, 1, 1), lambda b: (b, 0, 0)),
    ]
    return pl.pallas_call(
        _mix_kernel,
        grid=(B,),
        in_specs=bs,
        out_specs=pl.BlockSpec((1, N, PRED), lambda b: (b, 0, 0)),
        out_shape=jax.ShapeDtypeStruct((B, N, PRED), F32),
    )(hflat, mix_w.T, mix_b[None], stdev, mean)


def _adp_kernel(a_ref, p_ref, out_ref):
    out_ref[...] = _dot_t(a_ref[...], p_ref[...])


def _adp_call(adaptive, p3):
    rows = N * L // 4
    return pl.pallas_call(
        _adp_kernel,
        grid=(4,),
        in_specs=[pl.BlockSpec((rows, D), lambda i: (i, 0)),
                  pl.BlockSpec((D, D), lambda i: (0, 0))],
        out_specs=pl.BlockSpec((rows, D), lambda i: (i, 0)),
        out_shape=jax.ShapeDtypeStruct((N * L, D), F32),
    )(adaptive, p3)


def kernel(x_enc, x_mark_enc, x_dec, x_mark_dec, params):
    x = x_enc
    mean = jnp.mean(x, axis=(1, 2), keepdims=True)
    xc = x - mean
    stdev = jnp.sqrt(jnp.mean(xc * xc, axis=(1, 2), keepdims=True) + EPS)
    xn = xc / stdev
    xr = jnp.swapaxes(xn, 1, 2)
    xp = jnp.pad(xr, ((0, 0), (0, 0), (1, 1)))
    trend = (xp[..., :-2] + xp[..., 1:-1] + xp[..., 2:]) / 3.0
    season = xr - trend

    m6 = x_mark_enc[:, :, 6]
    m2 = x_mark_enc[:, :, 2]
    tod_idx = (m6 * SLICE).astype(jnp.int32)[..., None]
    dow_idx = m2.astype(jnp.int32)[..., None]

    proj_w = params['proj_W']
    emb_w = params['emb_W']
    p01 = proj_w[:, :D] + proj_w[:, D:2 * D]
    ut = (proj_w[:, :D] @ emb_w[:, 0])[None]
    us = (proj_w[:, D:2 * D] @ emb_w[:, 0])[None]
    u6 = (p01 @ emb_w[:, 1])[None]
    u2 = (p01 @ emb_w[:, 2])[None]
    cvec = (p01 @ params['emb_b'] + params['proj_b'])[None]

    sp = dict(params['spatial_block'])
    sp['spatial_W_T'] = params['spatial_W'].T
    s_proj = _spatial_call(xr.reshape(B * N, L), sp, proj_w)

    adaptive_p = _adp_call(params['adaptive'], proj_w[:, 2 * D:3 * D])
    adaptive_p = adaptive_p.reshape(N, L, D)

    h = _embed_call(trend, season, m6[..., None], m2[..., None],
                    tod_idx, dow_idx, params, (ut, us, u6, u2, cvec),
                    adaptive_p, s_proj.reshape(B, N, 1, D))

    h = h.reshape(B * N, L, D)
    for blk in params['blocks']:
        h = _block_call(h, blk)

    y = _mix_call(h.reshape(B, N, L * D), params['mix_W'], params['mix_b'],
                  stdev.reshape(B, 1, 1), mean.reshape(B, 1, 1))
    out = jnp.swapaxes(y, 1, 2)
    return out


# speedup vs baseline: 2.4332x; 1.0370x over previous
import functools

import jax
import jax.numpy as jnp
from jax.experimental import pallas as pl

B, L, N, C = 4, 96, 170, 1
D, H, DFF, E, TOPK = 128, 8, 512, 4, 2
NLAYERS = 3
SLICE = 288
PRED = 96
EPS = 1e-5
DH = D // H

F32 = jnp.float32


def _dot(a, b):
    return jax.lax.dot_general(a, b, (((1,), (0,)), ((), ())),
                               preferred_element_type=F32)


def _dot_t(a, b):
    return jax.lax.dot_general(a, b, (((1,), (1,)), ((), ())),
                               preferred_element_type=F32)


BF16 = jnp.bfloat16


def _dotb(a, b):
    return jax.lax.dot_general(a.astype(BF16), b.astype(BF16),
                               (((1,), (0,)), ((), ())),
                               preferred_element_type=F32)


def _ln(x, g, b):
    mu = jnp.mean(x, axis=-1, keepdims=True)
    xc = x - mu
    var = jnp.mean(xc * xc, axis=-1, keepdims=True)
    return xc * jax.lax.rsqrt(var + EPS) * g + b


def _top2_weights(logits):
    cols = [logits[:, e:e + 1] for e in range(E)]
    sels = []
    for e in range(E):
        rank = None
        for j in range(E):
            if j == e:
                continue
            gt = cols[j] > cols[e]
            if j < e:
                gt = gt | (cols[j] == cols[e])
            r = gt.astype(F32)
            rank = r if rank is None else rank + r
        sels.append(rank < 2.0)
    neg = jnp.float32(-1e30)
    m = None
    for e in range(E):
        v = jnp.where(sels[e], cols[e], neg)
        m = v if m is None else jnp.maximum(m, v)
    ws = []
    z = None
    for e in range(E):
        w = jnp.exp(cols[e] - m) * sels[e].astype(F32)
        ws.append(w)
        z = w if z is None else z + w
    inv = 1.0 / z
    return [w * inv for w in ws]


def _moe_dense(x, gate_w, gate_b, w1t, b1, w2t, b2):
    logits = _dot_t(x, gate_w) + gate_b
    ws = _top2_weights(logits)
    xb = x.astype(BF16)
    acc = None
    for e in range(E):
        h = jnp.maximum(_dotb(xb, w1t[e]) + b1[e:e + 1, :], 0.0)
        y = _dotb(h, w2t[e]) + b2[e:e + 1, :]
        y = y * ws[e]
        acc = y if acc is None else acc + y
    return acc


def _spatial_kernel(xs_ref, spwt_ref, wvt_ref, bv_ref, wot_ref, bo_ref,
                    ln2g_ref, ln2b_ref, ln3g_ref, ln3b_ref,
                    gw_ref, gb_ref, w1t_ref, b1_ref, w2t_ref, b2_ref,
                    p6_ref, out_ref):
    se = _dot(xs_ref[...], spwt_ref[...])
    a = _dot(se, wvt_ref[...]) + bv_ref[...]
    a = _dot(a, wot_ref[...]) + bo_ref[...]
    x1 = _ln(se + a, ln2g_ref[...], ln2b_ref[...])
    f = _moe_dense(x1, gw_ref[...], gb_ref[...], w1t_ref, b1_ref[...],
                   w2t_ref, b2_ref[...])
    sp = _ln(x1 + f, ln3g_ref[...], ln3b_ref[...])
    out_ref[...] = _dot_t(sp, p6_ref[...])


def _spatial_call(xs, p, proj_w):
    ap = p['attn']
    mp = p['moe']
    w1t = jnp.stack([e['W1'].T for e in mp['experts']])
    b1 = jnp.stack([e['b1'] for e in mp['experts']])
    w2t = jnp.stack([e['W2'].T for e in mp['experts']])
    b2 = jnp.stack([e['b2'] for e in mp['experts']])
    args = (xs, p['spatial_W_T'], ap['Wv'].T, ap['bv'][None], ap['Wo'].T,
            ap['bo'][None], p['ln2_g'][None], p['ln2_b'][None],
            p['ln3_g'][None], p['ln3_b'][None],
            mp['gate_W'], mp['gate_b'][None], w1t, b1, w2t, b2,
            proj_w[:, 5 * D:6 * D])
    return pl.pallas_call(
        _spatial_kernel,
        out_shape=jax.ShapeDtypeStruct((B * N, D), F32),
    )(*args)


NT = 34


def _embed_kernel(trend_ref, season_ref, m6_ref, m2_ref, tod_idx_ref,
                  dow_idx_ref, tod_tab_ref, dow_tab_ref, p4_ref, p5_ref,
                  ut_ref, us_ref, u6_ref, u2_ref, cvec_ref,
                  adp_ref, s_ref, out_ref):
    ut = ut_ref[...].reshape(1, 1, D)
    us = us_ref[...].reshape(1, 1, D)
    u6 = u6_ref[...]
    u2 = u2_ref[...]
    h = trend_ref[0] * ut + season_ref[0] * us
    bl = m6_ref[0] * u6 + m2_ref[0] * u2 + cvec_ref[...]
    tp = _dot_t(tod_tab_ref[...], p4_ref[...])
    dp = _dot_t(dow_tab_ref[...], p5_ref[...])
    ti = tod_idx_ref[0]
    di = dow_idx_ref[0]
    oh_t = (jax.lax.broadcasted_iota(jnp.int32, (L, SLICE), 1) == ti
            ).astype(F32)
    oh_d = (jax.lax.broadcasted_iota(jnp.int32, (L, 7), 1) == di
            ).astype(F32)
    bl = bl + _dot(oh_t, tp) + _dot(oh_d, dp)
    h = h + bl[None, :, :]
    h = h + adp_ref[...]
    h = h + jnp.broadcast_to(s_ref[0], (NT, L, D))
    out_ref[0] = h


def _embed_call(trend, season, m6, m2, tod_idx, dow_idx, params, folded,
                adaptive_p, s_proj):
    proj_w = params['proj_W']
    grid = (B, N // NT)
    bs = [
        pl.BlockSpec((1, NT, L, 1), lambda b, j: (b, j, 0, 0)),
        pl.BlockSpec((1, NT, L, 1), lambda b, j: (b, j, 0, 0)),
        pl.BlockSpec((1, L, 1), lambda b, j: (b, 0, 0)),
        pl.BlockSpec((1, L, 1), lambda b, j: (b, 0, 0)),
        pl.BlockSpec((1, L, 1), lambda b, j: (b, 0, 0)),
        pl.BlockSpec((1, L, 1), lambda b, j: (b, 0, 0)),
        pl.BlockSpec((SLICE, D), lambda b, j: (0, 0)),
        pl.BlockSpec((7, D), lambda b, j: (0, 0)),
        pl.BlockSpec((D, D), lambda b, j: (0, 0)),
        pl.BlockSpec((D, D), lambda b, j: (0, 0)),
        pl.BlockSpec((1, D), lambda b, j: (0, 0)),
        pl.BlockSpec((1, D), lambda b, j: (0, 0)),
        pl.BlockSpec((1, D), lambda b, j: (0, 0)),
        pl.BlockSpec((1, D), lambda b, j: (0, 0)),
        pl.BlockSpec((1, D), lambda b, j: (0, 0)),
        pl.BlockSpec((NT, L, D), lambda b, j: (j, 0, 0)),
        pl.BlockSpec((1, NT, 1, D), lambda b, j: (b, j, 0, 0)),
    ]
    ut, us, u6, u2, cvec = folded
    return pl.pallas_call(
        _embed_kernel,
        grid=grid,
        in_specs=bs,
        out_specs=pl.BlockSpec((1, NT, L, D), lambda b, j: (b, j, 0, 0)),
        out_shape=jax.ShapeDtypeStruct((B, N, L, D), F32),
    )(trend[..., None], season[..., None], m6, m2, tod_idx, dow_idx,
      params['tod_table'], params['dow_table'],
      proj_w[:, 3 * D:4 * D], proj_w[:, 4 * D:5 * D],
      ut, us, u6, u2, cvec, adaptive_p, s_proj)


G = 20


def _block_kernel(x_ref, wqt_ref, bq_ref, wkt_ref, bk_ref, wvt_ref, bv_ref,
                  wot_ref, bo_ref, ln2g_ref, ln2b_ref, ln3g_ref, ln3b_ref,
                  gw_ref, gb_ref, w1t_ref, b1_ref, w2t_ref, b2_ref, out_ref):
    x3 = x_ref[...]
    x = x3.reshape(G * L, D)
    xb = x.astype(BF16)
    q = _dotb(xb, wqt_ref[...]) + bq_ref[...]
    k = _dotb(xb, wkt_ref[...]) + bk_ref[...]
    v = _dotb(xb, wvt_ref[...]) + bv_ref[...]
    scale = 1.0 / (DH ** 0.5)
    heads = []
    for h in range(H):
        qh = (q[:, h * DH:(h + 1) * DH] * scale).reshape(G, L, DH)
        kh = k[:, h * DH:(h + 1) * DH].reshape(G, L, DH)
        vh = v[:, h * DH:(h + 1) * DH].reshape(G, L, DH)
        s = jax.lax.dot_general(qh.astype(BF16), kh.astype(BF16),
                                (((2,), (2,)), ((0,), (0,))),
                                preferred_element_type=F32)
        s = s - jnp.max(s, axis=-1, keepdims=True)
        p = jnp.exp(s)
        p = p / jnp.sum(p, axis=-1, keepdims=True)
        o = jax.lax.dot_general(p.astype(BF16), vh.astype(BF16),
                                (((2,), (1,)), ((0,), (0,))),
                                preferred_element_type=F32)
        heads.append(o.reshape(G * L, DH))
    a = jnp.concatenate(heads, axis=1)
    a = _dotb(a, wot_ref[...]) + bo_ref[...]
    x1 = _ln(x + a, ln2g_ref[...], ln2b_ref[...])
    f = _moe_dense(x1, gw_ref[...], gb_ref[...], w1t_ref, b1_ref[...],
                   w2t_ref, b2_ref[...])
    x2 = _ln(x1 + f, ln3g_ref[...], ln3b_ref[...])
    out_ref[...] = x2.reshape(G, L, D)


def _block_call(h, p):
    ap = p['attn']
    mp = p['moe']
    w1t = jnp.stack([e['W1'].T for e in mp['experts']])
    b1 = jnp.stack([e['b1'] for e in mp['experts']])
    w2t = jnp.stack([e['W2'].T for e in mp['experts']])
    b2 = jnp.stack([e['b2'] for e in mp['experts']])
    full = lambda shape: pl.BlockSpec(shape, lambda i: tuple(0 for _ in shape))
    bs = [pl.BlockSpec((G, L, D), lambda i: (i, 0, 0)),
          full((D, D)), full((1, D)), full((D, D)), full((1, D)),
          full((D, D)), full((1, D)), full((D, D)), full((1, D)),
          full((1, D)), full((1, D)), full((1, D)), full((1, D)),
          full((E, D)), full((1, E)), full((E, D, DFF)), full((E, DFF)),
          full((E, DFF, D)), full((E, D))]
    return pl.pallas_call(
        _block_kernel,
        grid=(B * N // G,),
        in_specs=bs,
        out_specs=pl.BlockSpec((G, L, D), lambda i: (i, 0, 0)),
        out_shape=jax.ShapeDtypeStruct((B * N, L, D), F32),
    )(h, ap['Wq'].T.astype(BF16), ap['bq'][None], ap['Wk'].T.astype(BF16),
      ap['bk'][None], ap['Wv'].T.astype(BF16), ap['bv'][None],
      ap['Wo'].T.astype(BF16), ap['bo'][None],
      p['ln2_g'][None], p['ln2_b'][None], p['ln3_g'][None], p['ln3_b'][None],
      mp['gate_W'], mp['gate_b'][None], w1t.astype(BF16), b1,
      w2t.astype(BF16), b2)


def _mix_kernel(h_ref, wt_ref, b_ref, st_ref, mu_ref, out_ref):
    y = _dot(h_ref[0], wt_ref[...]) + b_ref[...]
    out_ref[0] = y * st_ref[0, 0, 0] + mu_ref[0, 0, 0]


def _mix_call(hflat, mix_w, mix_b, stdev, mean):
    bs = [
        pl.BlockSpec((1, N, L * D), lambda b: (b, 0, 0)),
        pl.BlockSpec((L * D, PRED), lambda b: (0, 0)),
        pl.BlockSpec((1, PRED), lambda b: (0, 0)),
        pl.BlockSpec((1, 1, 1), lambda b: (b, 0, 0)),
        pl.BlockSpec((1, 1, 1), lambda b: (b, 0, 0)),
    ]
    return pl.pallas_call(
        _mix_kernel,
        grid=(B,),
        in_specs=bs,
        out_specs=pl.BlockSpec((1, N, PRED), lambda b: (b, 0, 0)),
        out_shape=jax.ShapeDtypeStruct((B, N, PRED), F32),
    )(hflat, mix_w.T, mix_b[None], stdev, mean)


def _adp_kernel(a_ref, p_ref, out_ref):
    out_ref[...] = _dot_t(a_ref[...], p_ref[...])


def _adp_call(adaptive, p3):
    rows = N * L // 4
    return pl.pallas_call(
        _adp_kernel,
        grid=(4,),
        in_specs=[pl.BlockSpec((rows, D), lambda i: (i, 0)),
                  pl.BlockSpec((D, D), lambda i: (0, 0))],
        out_specs=pl.BlockSpec((rows, D), lambda i: (i, 0)),
        out_shape=jax.ShapeDtypeStruct((N * L, D), F32),
    )(adaptive, p3)


def kernel(x_enc, x_mark_enc, x_dec, x_mark_dec, params):
    x = x_enc
    mean = jnp.mean(x, axis=(1, 2), keepdims=True)
    xc = x - mean
    stdev = jnp.sqrt(jnp.mean(xc * xc, axis=(1, 2), keepdims=True) + EPS)
    xn = xc / stdev
    xr = jnp.swapaxes(xn, 1, 2)
    xp = jnp.pad(xr, ((0, 0), (0, 0), (1, 1)))
    trend = (xp[..., :-2] + xp[..., 1:-1] + xp[..., 2:]) / 3.0
    season = xr - trend

    m6 = x_mark_enc[:, :, 6]
    m2 = x_mark_enc[:, :, 2]
    tod_idx = (m6 * SLICE).astype(jnp.int32)[..., None]
    dow_idx = m2.astype(jnp.int32)[..., None]

    proj_w = params['proj_W']
    emb_w = params['emb_W']
    p01 = proj_w[:, :D] + proj_w[:, D:2 * D]
    ut = (proj_w[:, :D] @ emb_w[:, 0])[None]
    us = (proj_w[:, D:2 * D] @ emb_w[:, 0])[None]
    u6 = (p01 @ emb_w[:, 1])[None]
    u2 = (p01 @ emb_w[:, 2])[None]
    cvec = (p01 @ params['emb_b'] + params['proj_b'])[None]

    sp = dict(params['spatial_block'])
    sp['spatial_W_T'] = params['spatial_W'].T
    s_proj = _spatial_call(xr.reshape(B * N, L), sp, proj_w)

    adaptive_p = _adp_call(params['adaptive'], proj_w[:, 2 * D:3 * D])
    adaptive_p = adaptive_p.reshape(N, L, D)

    h = _embed_call(trend, season, m6[..., None], m2[..., None],
                    tod_idx, dow_idx, params, (ut, us, u6, u2, cvec),
                    adaptive_p, s_proj.reshape(B, N, 1, D))

    h = h.reshape(B * N, L, D)
    for blk in params['blocks']:
        h = _block_call(h, blk)

    y = _mix_call(h.reshape(B, N, L * D), params['mix_W'], params['mix_b'],
                  stdev.reshape(B, 1, 1), mean.reshape(B, 1, 1))
    out = jnp.swapaxes(y, 1, 2)
    return out


# speedup vs baseline: 2.5917x; 1.0651x over previous
import functools

import jax
import jax.numpy as jnp
from jax.experimental import pallas as pl

B, L, N, C = 4, 96, 170, 1
D, H, DFF, E, TOPK = 128, 8, 512, 4, 2
NLAYERS = 3
SLICE = 288
PRED = 96
EPS = 1e-5
DH = D // H

F32 = jnp.float32


def _dot(a, b):
    return jax.lax.dot_general(a, b, (((1,), (0,)), ((), ())),
                               preferred_element_type=F32)


def _dot_t(a, b):
    return jax.lax.dot_general(a, b, (((1,), (1,)), ((), ())),
                               preferred_element_type=F32)


BF16 = jnp.bfloat16


def _dotb(a, b):
    return jax.lax.dot_general(a.astype(BF16), b.astype(BF16),
                               (((1,), (0,)), ((), ())),
                               preferred_element_type=F32)


def _ln(x, g, b):
    mu = jnp.mean(x, axis=-1, keepdims=True)
    xc = x - mu
    var = jnp.mean(xc * xc, axis=-1, keepdims=True)
    return xc * jax.lax.rsqrt(var + EPS) * g + b


def _row_sum_bc(x, ones_bf):
    hi = x.astype(BF16)
    lo = (x - hi.astype(F32)).astype(BF16)
    return (jax.lax.dot_general(hi, ones_bf, (((1,), (0,)), ((), ())),
                                preferred_element_type=F32)
            + jax.lax.dot_general(lo, ones_bf, (((1,), (0,)), ((), ())),
                                  preferred_element_type=F32))


def _ln_mxu(x, g, b, ones_bf):
    mu = _row_sum_bc(x, ones_bf) * (1.0 / D)
    xc = x - mu
    var = _row_sum_bc(xc * xc, ones_bf) * (1.0 / D)
    return xc * jax.lax.rsqrt(var + EPS) * g + b


def _top2_weights(logits):
    cols = [logits[:, e:e + 1] for e in range(E)]
    sels = []
    for e in range(E):
        rank = None
        for j in range(E):
            if j == e:
                continue
            gt = cols[j] > cols[e]
            if j < e:
                gt = gt | (cols[j] == cols[e])
            r = gt.astype(F32)
            rank = r if rank is None else rank + r
        sels.append(rank < 2.0)
    neg = jnp.float32(-1e30)
    m = None
    for e in range(E):
        v = jnp.where(sels[e], cols[e], neg)
        m = v if m is None else jnp.maximum(m, v)
    ws = []
    z = None
    for e in range(E):
        w = jnp.exp(cols[e] - m) * sels[e].astype(F32)
        ws.append(w)
        z = w if z is None else z + w
    inv = 1.0 / z
    return [w * inv for w in ws]


def _moe_dense(x, gate_w, gate_b, w1t, b1, w2t, b2):
    logits = _dot_t(x, gate_w) + gate_b
    ws = _top2_weights(logits)
    xb = x.astype(BF16)
    acc = None
    for e in range(E):
        h = jnp.maximum(_dotb(xb, w1t[e]) + b1[e:e + 1, :], 0.0)
        y = _dotb(h, w2t[e]) + b2[e:e + 1, :]
        y = y * ws[e]
        acc = y if acc is None else acc + y
    return acc


def _spatial_kernel(xs_ref, spwt_ref, wvt_ref, bv_ref, wot_ref, bo_ref,
                    ln2g_ref, ln2b_ref, ln3g_ref, ln3b_ref,
                    gw_ref, gb_ref, w1t_ref, b1_ref, w2t_ref, b2_ref,
                    p6_ref, out_ref):
    se = _dot(xs_ref[...], spwt_ref[...])
    a = _dot(se, wvt_ref[...]) + bv_ref[...]
    a = _dot(a, wot_ref[...]) + bo_ref[...]
    x1 = _ln(se + a, ln2g_ref[...], ln2b_ref[...])
    f = _moe_dense(x1, gw_ref[...], gb_ref[...], w1t_ref, b1_ref[...],
                   w2t_ref, b2_ref[...])
    sp = _ln(x1 + f, ln3g_ref[...], ln3b_ref[...])
    out_ref[...] = _dot_t(sp, p6_ref[...])


def _spatial_call(xs, p, proj_w):
    ap = p['attn']
    mp = p['moe']
    w1t = jnp.stack([e['W1'].T for e in mp['experts']])
    b1 = jnp.stack([e['b1'] for e in mp['experts']])
    w2t = jnp.stack([e['W2'].T for e in mp['experts']])
    b2 = jnp.stack([e['b2'] for e in mp['experts']])
    args = (xs, p['spatial_W_T'], ap['Wv'].T, ap['bv'][None], ap['Wo'].T,
            ap['bo'][None], p['ln2_g'][None], p['ln2_b'][None],
            p['ln3_g'][None], p['ln3_b'][None],
            mp['gate_W'], mp['gate_b'][None], w1t, b1, w2t, b2,
            proj_w[:, 5 * D:6 * D])
    return pl.pallas_call(
        _spatial_kernel,
        out_shape=jax.ShapeDtypeStruct((B * N, D), F32),
    )(*args)


NT = 34


def _embed_kernel(trend_ref, season_ref, m6_ref, m2_ref, tod_idx_ref,
                  dow_idx_ref, tod_tab_ref, dow_tab_ref, p4_ref, p5_ref,
                  ut_ref, us_ref, u6_ref, u2_ref, cvec_ref,
                  adp_ref, s_ref, out_ref):
    ut = ut_ref[...].reshape(1, 1, D)
    us = us_ref[...].reshape(1, 1, D)
    u6 = u6_ref[...]
    u2 = u2_ref[...]
    h = trend_ref[0] * ut + season_ref[0] * us
    bl = m6_ref[0] * u6 + m2_ref[0] * u2 + cvec_ref[...]
    tp = _dot_t(tod_tab_ref[...], p4_ref[...])
    dp = _dot_t(dow_tab_ref[...], p5_ref[...])
    ti = tod_idx_ref[0]
    di = dow_idx_ref[0]
    oh_t = (jax.lax.broadcasted_iota(jnp.int32, (L, SLICE), 1) == ti
            ).astype(F32)
    oh_d = (jax.lax.broadcasted_iota(jnp.int32, (L, 7), 1) == di
            ).astype(F32)
    bl = bl + _dot(oh_t, tp) + _dot(oh_d, dp)
    h = h + bl[None, :, :]
    h = h + adp_ref[...]
    h = h + jnp.broadcast_to(s_ref[0], (NT, L, D))
    out_ref[0] = h


def _embed_call(trend, season, m6, m2, tod_idx, dow_idx, params, folded,
                adaptive_p, s_proj):
    proj_w = params['proj_W']
    grid = (B, N // NT)
    bs = [
        pl.BlockSpec((1, NT, L, 1), lambda b, j: (b, j, 0, 0)),
        pl.BlockSpec((1, NT, L, 1), lambda b, j: (b, j, 0, 0)),
        pl.BlockSpec((1, L, 1), lambda b, j: (b, 0, 0)),
        pl.BlockSpec((1, L, 1), lambda b, j: (b, 0, 0)),
        pl.BlockSpec((1, L, 1), lambda b, j: (b, 0, 0)),
        pl.BlockSpec((1, L, 1), lambda b, j: (b, 0, 0)),
        pl.BlockSpec((SLICE, D), lambda b, j: (0, 0)),
        pl.BlockSpec((7, D), lambda b, j: (0, 0)),
        pl.BlockSpec((D, D), lambda b, j: (0, 0)),
        pl.BlockSpec((D, D), lambda b, j: (0, 0)),
        pl.BlockSpec((1, D), lambda b, j: (0, 0)),
        pl.BlockSpec((1, D), lambda b, j: (0, 0)),
        pl.BlockSpec((1, D), lambda b, j: (0, 0)),
        pl.BlockSpec((1, D), lambda b, j: (0, 0)),
        pl.BlockSpec((1, D), lambda b, j: (0, 0)),
        pl.BlockSpec((NT, L, D), lambda b, j: (j, 0, 0)),
        pl.BlockSpec((1, NT, 1, D), lambda b, j: (b, j, 0, 0)),
    ]
    ut, us, u6, u2, cvec = folded
    return pl.pallas_call(
        _embed_kernel,
        grid=grid,
        in_specs=bs,
        out_specs=pl.BlockSpec((1, NT, L, D), lambda b, j: (b, j, 0, 0)),
        out_shape=jax.ShapeDtypeStruct((B, N, L, D), F32),
    )(trend[..., None], season[..., None], m6, m2, tod_idx, dow_idx,
      params['tod_table'], params['dow_table'],
      proj_w[:, 3 * D:4 * D], proj_w[:, 4 * D:5 * D],
      ut, us, u6, u2, cvec, adaptive_p, s_proj)


G = 20


def _block_kernel(x_ref, wqt_ref, bq_ref, wkt_ref, bk_ref, wvt_ref, bv_ref,
                  wot_ref, bo_ref, ln2g_ref, ln2b_ref, ln3g_ref, ln3b_ref,
                  gw_ref, gb_ref, w1c_ref, b1c_ref, w2c_ref, b2s_ref,
                  ones_ref, ones96_ref, out_ref):
    x3 = x_ref[...]
    x = x3.reshape(G * L, D)
    xb = x.astype(BF16)
    q = _dotb(xb, wqt_ref[...]) + bq_ref[...]
    k = _dotb(xb, wkt_ref[...]) + bk_ref[...]
    v = _dotb(xb, wvt_ref[...]) + bv_ref[...]
    scale = 1.0 / (DH ** 0.5)
    heads = []
    for h in range(H):
        qh = (q[:, h * DH:(h + 1) * DH] * scale).reshape(G, L, DH)
        kh = k[:, h * DH:(h + 1) * DH].reshape(G, L, DH)
        vh = v[:, h * DH:(h + 1) * DH].reshape(G, L, DH)
        s = jax.lax.dot_general(qh.astype(BF16), kh.astype(BF16),
                                (((2,), (2,)), ((0,), (0,))),
                                preferred_element_type=F32)
        p = jnp.exp(s).astype(BF16)
        o = jax.lax.dot_general(p, vh.astype(BF16),
                                (((2,), (1,)), ((0,), (0,))),
                                preferred_element_type=F32)
        den = jax.lax.dot_general(p.reshape(G * L, L), ones96_ref[...],
                                  (((1,), (0,)), ((), ())),
                                  preferred_element_type=F32)
        heads.append(o.reshape(G * L, DH) / den)
    a = jnp.concatenate(heads, axis=1)
    a = _dotb(a, wot_ref[...]) + bo_ref[...]
    ones_bf = ones_ref[...]
    x1 = _ln_mxu(x + a, ln2g_ref[...], ln2b_ref[...], ones_bf)
    logits = jax.lax.dot_general(x1, gw_ref[...], (((1,), (1,)), ((), ())),
                                 preferred_element_type=F32,
                                 precision=jax.lax.Precision.HIGHEST)
    logits = logits + gb_ref[...]
    ws = _top2_weights(logits)
    h1 = jnp.maximum(_dotb(x1.astype(BF16), w1c_ref[...]) + b1c_ref[...],
                     0.0).astype(BF16)
    parts = [h1[:, e * DFF:(e + 1) * DFF] * ws[e].astype(BF16)
             for e in range(E)]
    h1w = jnp.concatenate(parts, axis=1)
    f = _dotb(h1w, w2c_ref[...])
    w4 = jnp.concatenate(ws, axis=1)
    f = f + jax.lax.dot_general(w4, b2s_ref[...], (((1,), (0,)), ((), ())),
                                preferred_element_type=F32)
    x2 = _ln_mxu(x1 + f, ln3g_ref[...], ln3b_ref[...], ones_bf)
    out_ref[...] = x2.reshape(G, L, D)


def _block_call(h, p):
    ap = p['attn']
    mp = p['moe']
    w1c = jnp.concatenate([e['W1'].T for e in mp['experts']], axis=1)
    b1c = jnp.concatenate([e['b1'] for e in mp['experts']])[None]
    w2c = jnp.concatenate([e['W2'].T for e in mp['experts']], axis=0)
    b2s = jnp.stack([e['b2'] for e in mp['experts']])
    ones_bf = jnp.ones((D, D), BF16)
    ones96 = jnp.ones((L, DH), BF16)
    full = lambda shape: pl.BlockSpec(shape, lambda i: tuple(0 for _ in shape))
    bs = [pl.BlockSpec((G, L, D), lambda i: (i, 0, 0)),
          full((D, D)), full((1, D)), full((D, D)), full((1, D)),
          full((D, D)), full((1, D)), full((D, D)), full((1, D)),
          full((1, D)), full((1, D)), full((1, D)), full((1, D)),
          full((E, D)), full((1, E)), full((D, E * DFF)), full((1, E * DFF)),
          full((E * DFF, D)), full((E, D)), full((D, D)), full((L, DH))]
    return pl.pallas_call(
        _block_kernel,
        grid=(B * N // G,),
        in_specs=bs,
        out_specs=pl.BlockSpec((G, L, D), lambda i: (i, 0, 0)),
        out_shape=jax.ShapeDtypeStruct((B * N, L, D), F32),
    )(h, ap['Wq'].T.astype(BF16), ap['bq'][None], ap['Wk'].T.astype(BF16),
      ap['bk'][None], ap['Wv'].T.astype(BF16), ap['bv'][None],
      ap['Wo'].T.astype(BF16), ap['bo'][None],
      p['ln2_g'][None], p['ln2_b'][None], p['ln3_g'][None], p['ln3_b'][None],
      mp['gate_W'], mp['gate_b'][None], w1c.astype(BF16), b1c,
      w2c.astype(BF16), b2s, ones_bf, ones96)


def _mix_kernel(h_ref, wt_ref, b_ref, st_ref, mu_ref, out_ref):
    y = _dot(h_ref[0], wt_ref[...]) + b_ref[...]
    out_ref[0] = y * st_ref[0, 0, 0] + mu_ref[0, 0, 0]


def _mix_call(hflat, mix_w, mix_b, stdev, mean):
    bs = [
        pl.BlockSpec((1, N, L * D), lambda b: (b, 0, 0)),
        pl.BlockSpec((L * D, PRED), lambda b: (0, 0)),
        pl.BlockSpec((1, PRED), lambda b: (0, 0)),
        pl.BlockSpec((1, 1, 1), lambda b: (b, 0, 0)),
        pl.BlockSpec((1, 1, 1), lambda b: (b, 0, 0)),
    ]
    return pl.pallas_call(
        _mix_kernel,
        grid=(B,),
        in_specs=bs,
        out_specs=pl.BlockSpec((1, N, PRED), lambda b: (b, 0, 0)),
        out_shape=jax.ShapeDtypeStruct((B, N, PRED), F32),
    )(hflat, mix_w.T, mix_b[None], stdev, mean)


def _adp_kernel(a_ref, p_ref, out_ref):
    out_ref[...] = _dot_t(a_ref[...], p_ref[...])


def _adp_call(adaptive, p3):
    rows = N * L // 4
    return pl.pallas_call(
        _adp_kernel,
        grid=(4,),
        in_specs=[pl.BlockSpec((rows, D), lambda i: (i, 0)),
                  pl.BlockSpec((D, D), lambda i: (0, 0))],
        out_specs=pl.BlockSpec((rows, D), lambda i: (i, 0)),
        out_shape=jax.ShapeDtypeStruct((N * L, D), F32),
    )(adaptive, p3)


def kernel(x_enc, x_mark_enc, x_dec, x_mark_dec, params):
    x = x_enc
    mean = jnp.mean(x, axis=(1, 2), keepdims=True)
    xc = x - mean
    stdev = jnp.sqrt(jnp.mean(xc * xc, axis=(1, 2), keepdims=True) + EPS)
    xn = xc / stdev
    xr = jnp.swapaxes(xn, 1, 2)
    xp = jnp.pad(xr, ((0, 0), (0, 0), (1, 1)))
    trend = (xp[..., :-2] + xp[..., 1:-1] + xp[..., 2:]) / 3.0
    season = xr - trend

    m6 = x_mark_enc[:, :, 6]
    m2 = x_mark_enc[:, :, 2]
    tod_idx = (m6 * SLICE).astype(jnp.int32)[..., None]
    dow_idx = m2.astype(jnp.int32)[..., None]

    proj_w = params['proj_W']
    emb_w = params['emb_W']
    p01 = proj_w[:, :D] + proj_w[:, D:2 * D]
    ut = (proj_w[:, :D] @ emb_w[:, 0])[None]
    us = (proj_w[:, D:2 * D] @ emb_w[:, 0])[None]
    u6 = (p01 @ emb_w[:, 1])[None]
    u2 = (p01 @ emb_w[:, 2])[None]
    cvec = (p01 @ params['emb_b'] + params['proj_b'])[None]

    sp = dict(params['spatial_block'])
    sp['spatial_W_T'] = params['spatial_W'].T
    s_proj = _spatial_call(xr.reshape(B * N, L), sp, proj_w)

    adaptive_p = _adp_call(params['adaptive'], proj_w[:, 2 * D:3 * D])
    adaptive_p = adaptive_p.reshape(N, L, D)

    h = _embed_call(trend, season, m6[..., None], m2[..., None],
                    tod_idx, dow_idx, params, (ut, us, u6, u2, cvec),
                    adaptive_p, s_proj.reshape(B, N, 1, D))

    h = h.reshape(B * N, L, D)
    for blk in params['blocks']:
        h = _block_call(h, blk)

    y = _mix_call(h.reshape(B, N, L * D), params['mix_W'], params['mix_b'],
                  stdev.reshape(B, 1, 1), mean.reshape(B, 1, 1))
    out = jnp.swapaxes(y, 1, 2)
    return out


# speedup vs baseline: 2.8028x; 1.0815x over previous
import functools

import jax
import jax.numpy as jnp
from jax.experimental import pallas as pl

B, L, N, C = 4, 96, 170, 1
D, H, DFF, E, TOPK = 128, 8, 512, 4, 2
NLAYERS = 3
SLICE = 288
PRED = 96
EPS = 1e-5
DH = D // H

F32 = jnp.float32


def _dot(a, b):
    return jax.lax.dot_general(a, b, (((1,), (0,)), ((), ())),
                               preferred_element_type=F32)


def _dot_t(a, b):
    return jax.lax.dot_general(a, b, (((1,), (1,)), ((), ())),
                               preferred_element_type=F32)


BF16 = jnp.bfloat16


def _dotb(a, b):
    return jax.lax.dot_general(a.astype(BF16), b.astype(BF16),
                               (((1,), (0,)), ((), ())),
                               preferred_element_type=F32)


def _ln(x, g, b):
    mu = jnp.mean(x, axis=-1, keepdims=True)
    xc = x - mu
    var = jnp.mean(xc * xc, axis=-1, keepdims=True)
    return xc * jax.lax.rsqrt(var + EPS) * g + b


def _row_sum_bc(x, ones_bf):
    hi = x.astype(BF16)
    lo = (x - hi.astype(F32)).astype(BF16)
    return (jax.lax.dot_general(hi, ones_bf, (((1,), (0,)), ((), ())),
                                preferred_element_type=F32)
            + jax.lax.dot_general(lo, ones_bf, (((1,), (0,)), ((), ())),
                                  preferred_element_type=F32))


def _ln_mxu(x, g, b, ones_bf):
    mu = _row_sum_bc(x, ones_bf) * (1.0 / D)
    xc = x - mu
    var = _row_sum_bc(xc * xc, ones_bf) * (1.0 / D)
    return xc * jax.lax.rsqrt(var + EPS) * g + b


def _top2_weights_t(logits_t):
    cols = [logits_t[e:e + 1, :] for e in range(E)]
    sels = []
    for e in range(E):
        rank = None
        for j in range(E):
            if j == e:
                continue
            gt = cols[j] > cols[e]
            if j < e:
                gt = gt | (cols[j] == cols[e])
            r = gt.astype(F32)
            rank = r if rank is None else rank + r
        sels.append(rank < 2.0)
    neg = jnp.float32(-1e30)
    m = None
    for e in range(E):
        v = jnp.where(sels[e], cols[e], neg)
        m = v if m is None else jnp.maximum(m, v)
    ws = []
    z = None
    for e in range(E):
        w = jnp.exp(cols[e] - m) * sels[e].astype(F32)
        ws.append(w)
        z = w if z is None else z + w
    inv = 1.0 / z
    return jnp.concatenate([w * inv for w in ws], axis=0)


def _top2_weights(logits):
    cols = [logits[:, e:e + 1] for e in range(E)]
    sels = []
    for e in range(E):
        rank = None
        for j in range(E):
            if j == e:
                continue
            gt = cols[j] > cols[e]
            if j < e:
                gt = gt | (cols[j] == cols[e])
            r = gt.astype(F32)
            rank = r if rank is None else rank + r
        sels.append(rank < 2.0)
    neg = jnp.float32(-1e30)
    m = None
    for e in range(E):
        v = jnp.where(sels[e], cols[e], neg)
        m = v if m is None else jnp.maximum(m, v)
    ws = []
    z = None
    for e in range(E):
        w = jnp.exp(cols[e] - m) * sels[e].astype(F32)
        ws.append(w)
        z = w if z is None else z + w
    inv = 1.0 / z
    return [w * inv for w in ws]


def _moe_dense(x, gate_w, gate_b, w1t, b1, w2t, b2):
    logits = _dot_t(x, gate_w) + gate_b
    ws = _top2_weights(logits)
    xb = x.astype(BF16)
    acc = None
    for e in range(E):
        h = jnp.maximum(_dotb(xb, w1t[e]) + b1[e:e + 1, :], 0.0)
        y = _dotb(h, w2t[e]) + b2[e:e + 1, :]
        y = y * ws[e]
        acc = y if acc is None else acc + y
    return acc


def _spatial_kernel(xs_ref, spwt_ref, wvt_ref, bv_ref, wot_ref, bo_ref,
                    ln2g_ref, ln2b_ref, ln3g_ref, ln3b_ref,
                    gw_ref, gb_ref, w1t_ref, b1_ref, w2t_ref, b2_ref,
                    p6_ref, out_ref):
    se = _dot(xs_ref[...], spwt_ref[...])
    a = _dot(se, wvt_ref[...]) + bv_ref[...]
    a = _dot(a, wot_ref[...]) + bo_ref[...]
    x1 = _ln(se + a, ln2g_ref[...], ln2b_ref[...])
    f = _moe_dense(x1, gw_ref[...], gb_ref[...], w1t_ref, b1_ref[...],
                   w2t_ref, b2_ref[...])
    sp = _ln(x1 + f, ln3g_ref[...], ln3b_ref[...])
    out_ref[...] = _dot_t(sp, p6_ref[...])


def _spatial_call(xs, p, proj_w):
    ap = p['attn']
    mp = p['moe']
    w1t = jnp.stack([e['W1'].T for e in mp['experts']])
    b1 = jnp.stack([e['b1'] for e in mp['experts']])
    w2t = jnp.stack([e['W2'].T for e in mp['experts']])
    b2 = jnp.stack([e['b2'] for e in mp['experts']])
    args = (xs, p['spatial_W_T'], ap['Wv'].T, ap['bv'][None], ap['Wo'].T,
            ap['bo'][None], p['ln2_g'][None], p['ln2_b'][None],
            p['ln3_g'][None], p['ln3_b'][None],
            mp['gate_W'], mp['gate_b'][None], w1t, b1, w2t, b2,
            proj_w[:, 5 * D:6 * D])
    return pl.pallas_call(
        _spatial_kernel,
        out_shape=jax.ShapeDtypeStruct((B * N, D), F32),
    )(*args)


NT = 34


def _embed_kernel(trend_ref, season_ref, m6_ref, m2_ref, tod_idx_ref,
                  dow_idx_ref, tod_tab_ref, dow_tab_ref, p4_ref, p5_ref,
                  ut_ref, us_ref, u6_ref, u2_ref, cvec_ref,
                  adp_ref, s_ref, out_ref):
    ut = ut_ref[...].reshape(1, 1, D)
    us = us_ref[...].reshape(1, 1, D)
    u6 = u6_ref[...]
    u2 = u2_ref[...]
    h = trend_ref[0] * ut + season_ref[0] * us
    bl = m6_ref[0] * u6 + m2_ref[0] * u2 + cvec_ref[...]
    tp = _dot_t(tod_tab_ref[...], p4_ref[...])
    dp = _dot_t(dow_tab_ref[...], p5_ref[...])
    ti = tod_idx_ref[0]
    di = dow_idx_ref[0]
    oh_t = (jax.lax.broadcasted_iota(jnp.int32, (L, SLICE), 1) == ti
            ).astype(F32)
    oh_d = (jax.lax.broadcasted_iota(jnp.int32, (L, 7), 1) == di
            ).astype(F32)
    bl = bl + _dot(oh_t, tp) + _dot(oh_d, dp)
    h = h + bl[None, :, :]
    h = h + adp_ref[...]
    h = h + jnp.broadcast_to(s_ref[0], (NT, L, D))
    out_ref[0] = h


def _embed_call(trend, season, m6, m2, tod_idx, dow_idx, params, folded,
                adaptive_p, s_proj):
    proj_w = params['proj_W']
    grid = (B, N // NT)
    bs = [
        pl.BlockSpec((1, NT, L, 1), lambda b, j: (b, j, 0, 0)),
        pl.BlockSpec((1, NT, L, 1), lambda b, j: (b, j, 0, 0)),
        pl.BlockSpec((1, L, 1), lambda b, j: (b, 0, 0)),
        pl.BlockSpec((1, L, 1), lambda b, j: (b, 0, 0)),
        pl.BlockSpec((1, L, 1), lambda b, j: (b, 0, 0)),
        pl.BlockSpec((1, L, 1), lambda b, j: (b, 0, 0)),
        pl.BlockSpec((SLICE, D), lambda b, j: (0, 0)),
        pl.BlockSpec((7, D), lambda b, j: (0, 0)),
        pl.BlockSpec((D, D), lambda b, j: (0, 0)),
        pl.BlockSpec((D, D), lambda b, j: (0, 0)),
        pl.BlockSpec((1, D), lambda b, j: (0, 0)),
        pl.BlockSpec((1, D), lambda b, j: (0, 0)),
        pl.BlockSpec((1, D), lambda b, j: (0, 0)),
        pl.BlockSpec((1, D), lambda b, j: (0, 0)),
        pl.BlockSpec((1, D), lambda b, j: (0, 0)),
        pl.BlockSpec((NT, L, D), lambda b, j: (j, 0, 0)),
        pl.BlockSpec((1, NT, 1, D), lambda b, j: (b, j, 0, 0)),
    ]
    ut, us, u6, u2, cvec = folded
    return pl.pallas_call(
        _embed_kernel,
        grid=grid,
        in_specs=bs,
        out_specs=pl.BlockSpec((1, NT, L, D), lambda b, j: (b, j, 0, 0)),
        out_shape=jax.ShapeDtypeStruct((B, N, L, D), F32),
    )(trend[..., None], season[..., None], m6, m2, tod_idx, dow_idx,
      params['tod_table'], params['dow_table'],
      proj_w[:, 3 * D:4 * D], proj_w[:, 4 * D:5 * D],
      ut, us, u6, u2, cvec, adaptive_p, s_proj)


G = 20


def _one_block(x, l, wqt_ref, bq_ref, wkt_ref, bk_ref, wvt_ref, bv_ref,
               wot_ref, bo_ref, ln2g_ref, ln2b_ref, ln3g_ref, ln3b_ref,
               gw_ref, gb_ref, w1c_ref, b1c_ref, w2c_ref, b2s_ref,
               ones_bf, ones96):
    xb = x.astype(BF16)
    q = _dotb(xb, wqt_ref[l]) + bq_ref[l]
    k = _dotb(xb, wkt_ref[l]) + bk_ref[l]
    v = _dotb(xb, wvt_ref[l]) + bv_ref[l]
    heads = []
    for h in range(H):
        qh = q[:, h * DH:(h + 1) * DH].reshape(G, L, DH)
        kh = k[:, h * DH:(h + 1) * DH].reshape(G, L, DH)
        vh = v[:, h * DH:(h + 1) * DH].reshape(G, L, DH)
        s = jax.lax.dot_general(qh.astype(BF16), kh.astype(BF16),
                                (((2,), (2,)), ((0,), (0,))),
                                preferred_element_type=F32)
        p = jnp.exp(s).astype(BF16)
        o = jax.lax.dot_general(p, vh.astype(BF16),
                                (((2,), (1,)), ((0,), (0,))),
                                preferred_element_type=F32)
        den = jax.lax.dot_general(p.reshape(G * L, L), ones96,
                                  (((1,), (0,)), ((), ())),
                                  preferred_element_type=F32)
        heads.append(o.reshape(G * L, DH) / den)
    a = jnp.concatenate(heads, axis=1)
    a = _dotb(a, wot_ref[l]) + bo_ref[l]
    x1 = _ln_mxu(x + a, ln2g_ref[l], ln2b_ref[l], ones_bf)
    logits_t = jax.lax.dot_general(gw_ref[l], x1, (((1,), (1,)), ((), ())),
                                   preferred_element_type=F32,
                                   precision=jax.lax.Precision.HIGHEST)
    logits_t = logits_t + gb_ref[l]
    w_t = _top2_weights_t(logits_t)
    w4 = jnp.transpose(w_t).astype(BF16)
    h1 = jnp.maximum((_dotb(x1.astype(BF16), w1c_ref[l])
                      + b1c_ref[l]).astype(BF16), 0)
    parts = [h1[:, e * DFF:(e + 1) * DFF] * w4[:, e:e + 1]
             for e in range(E)]
    h1w = jnp.concatenate(parts, axis=1)
    f = _dotb(h1w, w2c_ref[l])
    f = f + jax.lax.dot_general(w4, b2s_ref[l], (((1,), (0,)), ((), ())),
                                preferred_element_type=F32)
    return _ln_mxu(x1 + f, ln3g_ref[l], ln3b_ref[l], ones_bf)


def _layers_kernel(x_ref, wqt_ref, bq_ref, wkt_ref, bk_ref, wvt_ref, bv_ref,
                   wot_ref, bo_ref, ln2g_ref, ln2b_ref, ln3g_ref, ln3b_ref,
                   gw_ref, gb_ref, w1c_ref, b1c_ref, w2c_ref, b2s_ref,
                   ones_ref, ones96_ref, out_ref):
    x = x_ref[...].reshape(G * L, D)
    ones_bf = ones_ref[...]
    ones96 = ones96_ref[...]
    for l in range(NLAYERS):
        x = _one_block(x, l, wqt_ref, bq_ref, wkt_ref, bk_ref, wvt_ref,
                       bv_ref, wot_ref, bo_ref, ln2g_ref, ln2b_ref,
                       ln3g_ref, ln3b_ref, gw_ref, gb_ref, w1c_ref,
                       b1c_ref, w2c_ref, b2s_ref, ones_bf, ones96)
    out_ref[...] = x.reshape(G, L, D)


def _layers_call(h, blocks):
    scale = 1.0 / (DH ** 0.5)
    wqt = jnp.stack([p['attn']['Wq'].T * scale for p in blocks]).astype(BF16)
    bq = jnp.stack([p['attn']['bq'][None] * scale for p in blocks])
    wkt = jnp.stack([p['attn']['Wk'].T for p in blocks]).astype(BF16)
    bk = jnp.stack([p['attn']['bk'][None] for p in blocks])
    wvt = jnp.stack([p['attn']['Wv'].T for p in blocks]).astype(BF16)
    bv = jnp.stack([p['attn']['bv'][None] for p in blocks])
    wot = jnp.stack([p['attn']['Wo'].T for p in blocks]).astype(BF16)
    bo = jnp.stack([p['attn']['bo'][None] for p in blocks])
    ln2g = jnp.stack([p['ln2_g'][None] for p in blocks])
    ln2b = jnp.stack([p['ln2_b'][None] for p in blocks])
    ln3g = jnp.stack([p['ln3_g'][None] for p in blocks])
    ln3b = jnp.stack([p['ln3_b'][None] for p in blocks])
    gw = jnp.stack([p['moe']['gate_W'] for p in blocks])
    gb = jnp.stack([p['moe']['gate_b'][:, None] for p in blocks])
    w1c = jnp.stack([
        jnp.concatenate([e['W1'].T for e in p['moe']['experts']], axis=1)
        for p in blocks]).astype(BF16)
    b1c = jnp.stack([
        jnp.concatenate([e['b1'] for e in p['moe']['experts']])[None]
        for p in blocks])
    w2c = jnp.stack([
        jnp.concatenate([e['W2'].T for e in p['moe']['experts']], axis=0)
        for p in blocks]).astype(BF16)
    b2s = jnp.stack([
        jnp.stack([e['b2'] for e in p['moe']['experts']]) for p in blocks])
    ones_bf = jnp.ones((D, D), BF16)
    ones96 = jnp.ones((L, DH), BF16)
    full = lambda shape: pl.BlockSpec(shape, lambda i: tuple(0 for _ in shape))
    KK = NLAYERS
    bs = [pl.BlockSpec((G, L, D), lambda i: (i, 0, 0)),
          full((KK, D, D)), full((KK, 1, D)), full((KK, D, D)),
          full((KK, 1, D)), full((KK, D, D)), full((KK, 1, D)),
          full((KK, D, D)), full((KK, 1, D)),
          full((KK, 1, D)), full((KK, 1, D)), full((KK, 1, D)),
          full((KK, 1, D)),
          full((KK, E, D)), full((KK, E, 1)), full((KK, D, E * DFF)),
          full((KK, 1, E * DFF)), full((KK, E * DFF, D)), full((KK, E, D)),
          full((D, D)), full((L, DH))]
    return pl.pallas_call(
        _layers_kernel,
        grid=(B * N // G,),
        in_specs=bs,
        out_specs=pl.BlockSpec((G, L, D), lambda i: (i, 0, 0)),
        out_shape=jax.ShapeDtypeStruct((B * N, L, D), F32),
    )(h, wqt, bq, wkt, bk, wvt, bv, wot, bo, ln2g, ln2b, ln3g, ln3b,
      gw, gb, w1c, b1c, w2c, b2s, ones_bf, ones96)


def _mix_kernel(h_ref, wt_ref, b_ref, st_ref, mu_ref, out_ref):
    y = _dot(h_ref[0], wt_ref[...]) + b_ref[...]
    out_ref[0] = y * st_ref[0, 0, 0] + mu_ref[0, 0, 0]


def _mix_call(hflat, mix_w, mix_b, stdev, mean):
    bs = [
        pl.BlockSpec((1, N, L * D), lambda b: (b, 0, 0)),
        pl.BlockSpec((L * D, PRED), lambda b: (0, 0)),
        pl.BlockSpec((1, PRED), lambda b: (0, 0)),
        pl.BlockSpec((1, 1, 1), lambda b: (b, 0, 0)),
        pl.BlockSpec((1, 1, 1), lambda b: (b, 0, 0)),
    ]
    return pl.pallas_call(
        _mix_kernel,
        grid=(B,),
        in_specs=bs,
        out_specs=pl.BlockSpec((1, N, PRED), lambda b: (b, 0, 0)),
        out_shape=jax.ShapeDtypeStruct((B, N, PRED), F32),
    )(hflat, mix_w.T, mix_b[None], stdev, mean)


def _adp_kernel(a_ref, p_ref, out_ref):
    out_ref[...] = _dot_t(a_ref[...], p_ref[...])


def _adp_call(adaptive, p3):
    rows = N * L // 4
    return pl.pallas_call(
        _adp_kernel,
        grid=(4,),
        in_specs=[pl.BlockSpec((rows, D), lambda i: (i, 0)),
                  pl.BlockSpec((D, D), lambda i: (0, 0))],
        out_specs=pl.BlockSpec((rows, D), lambda i: (i, 0)),
        out_shape=jax.ShapeDtypeStruct((N * L, D), F32),
    )(adaptive, p3)


def kernel(x_enc, x_mark_enc, x_dec, x_mark_dec, params):
    x = x_enc
    mean = jnp.mean(x, axis=(1, 2), keepdims=True)
    xc = x - mean
    stdev = jnp.sqrt(jnp.mean(xc * xc, axis=(1, 2), keepdims=True) + EPS)
    xn = xc / stdev
    xr = jnp.swapaxes(xn, 1, 2)
    xp = jnp.pad(xr, ((0, 0), (0, 0), (1, 1)))
    trend = (xp[..., :-2] + xp[..., 1:-1] + xp[..., 2:]) / 3.0
    season = xr - trend

    m6 = x_mark_enc[:, :, 6]
    m2 = x_mark_enc[:, :, 2]
    tod_idx = (m6 * SLICE).astype(jnp.int32)[..., None]
    dow_idx = m2.astype(jnp.int32)[..., None]

    proj_w = params['proj_W']
    emb_w = params['emb_W']
    p01 = proj_w[:, :D] + proj_w[:, D:2 * D]
    ut = (proj_w[:, :D] @ emb_w[:, 0])[None]
    us = (proj_w[:, D:2 * D] @ emb_w[:, 0])[None]
    u6 = (p01 @ emb_w[:, 1])[None]
    u2 = (p01 @ emb_w[:, 2])[None]
    cvec = (p01 @ params['emb_b'] + params['proj_b'])[None]

    sp = dict(params['spatial_block'])
    sp['spatial_W_T'] = params['spatial_W'].T
    s_proj = _spatial_call(xr.reshape(B * N, L), sp, proj_w)

    adaptive_p = _adp_call(params['adaptive'], proj_w[:, 2 * D:3 * D])
    adaptive_p = adaptive_p.reshape(N, L, D)

    h = _embed_call(trend, season, m6[..., None], m2[..., None],
                    tod_idx, dow_idx, params, (ut, us, u6, u2, cvec),
                    adaptive_p, s_proj.reshape(B, N, 1, D))

    h = _layers_call(h.reshape(B * N, L, D), params['blocks'])

    y = _mix_call(h.reshape(B, N, L * D), params['mix_W'], params['mix_b'],
                  stdev.reshape(B, 1, 1), mean.reshape(B, 1, 1))
    out = jnp.swapaxes(y, 1, 2)
    return out


# speedup vs baseline: 3.0400x; 1.0846x over previous
import functools

import jax
import jax.numpy as jnp
from jax.experimental import pallas as pl

B, L, N, C = 4, 96, 170, 1
D, H, DFF, E, TOPK = 128, 8, 512, 4, 2
NLAYERS = 3
SLICE = 288
PRED = 96
EPS = 1e-5
DH = D // H

F32 = jnp.float32


def _dot(a, b):
    return jax.lax.dot_general(a, b, (((1,), (0,)), ((), ())),
                               preferred_element_type=F32)


def _dot_t(a, b):
    return jax.lax.dot_general(a, b, (((1,), (1,)), ((), ())),
                               preferred_element_type=F32)


BF16 = jnp.bfloat16


def _dotb(a, b):
    return jax.lax.dot_general(a.astype(BF16), b.astype(BF16),
                               (((1,), (0,)), ((), ())),
                               preferred_element_type=F32)


def _ln(x, g, b):
    mu = jnp.mean(x, axis=-1, keepdims=True)
    xc = x - mu
    var = jnp.mean(xc * xc, axis=-1, keepdims=True)
    return xc * jax.lax.rsqrt(var + EPS) * g + b


def _row_sum_bc(x, ones_bf):
    hi = x.astype(BF16)
    lo = (x - hi.astype(F32)).astype(BF16)
    return (jax.lax.dot_general(hi, ones_bf, (((1,), (0,)), ((), ())),
                                preferred_element_type=F32)
            + jax.lax.dot_general(lo, ones_bf, (((1,), (0,)), ((), ())),
                                  preferred_element_type=F32))


def _ln_mxu(x, g, b, ones_bf):
    mu = _row_sum_bc(x, ones_bf) * (1.0 / D)
    xc = x - mu
    var = _row_sum_bc(xc * xc, ones_bf) * (1.0 / D)
    return xc * jax.lax.rsqrt(var + EPS) * g + b


def _top2_weights_t(logits_t):
    cols = [logits_t[e:e + 1, :] for e in range(E)]
    sels = []
    for e in range(E):
        rank = None
        for j in range(E):
            if j == e:
                continue
            gt = cols[j] > cols[e]
            if j < e:
                gt = gt | (cols[j] == cols[e])
            r = gt.astype(F32)
            rank = r if rank is None else rank + r
        sels.append(rank < 2.0)
    neg = jnp.float32(-1e30)
    m = None
    for e in range(E):
        v = jnp.where(sels[e], cols[e], neg)
        m = v if m is None else jnp.maximum(m, v)
    ws = []
    z = None
    for e in range(E):
        w = jnp.exp(cols[e] - m) * sels[e].astype(F32)
        ws.append(w)
        z = w if z is None else z + w
    inv = 1.0 / z
    return jnp.concatenate([w * inv for w in ws], axis=0)


def _top2_weights(logits):
    cols = [logits[:, e:e + 1] for e in range(E)]
    sels = []
    for e in range(E):
        rank = None
        for j in range(E):
            if j == e:
                continue
            gt = cols[j] > cols[e]
            if j < e:
                gt = gt | (cols[j] == cols[e])
            r = gt.astype(F32)
            rank = r if rank is None else rank + r
        sels.append(rank < 2.0)
    neg = jnp.float32(-1e30)
    m = None
    for e in range(E):
        v = jnp.where(sels[e], cols[e], neg)
        m = v if m is None else jnp.maximum(m, v)
    ws = []
    z = None
    for e in range(E):
        w = jnp.exp(cols[e] - m) * sels[e].astype(F32)
        ws.append(w)
        z = w if z is None else z + w
    inv = 1.0 / z
    return [w * inv for w in ws]


def _moe_dense(x, gate_w, gate_b, w1t, b1, w2t, b2):
    logits = _dot_t(x, gate_w) + gate_b
    ws = _top2_weights(logits)
    xb = x.astype(BF16)
    acc = None
    for e in range(E):
        h = jnp.maximum(_dotb(xb, w1t[e]) + b1[e:e + 1, :], 0.0)
        y = _dotb(h, w2t[e]) + b2[e:e + 1, :]
        y = y * ws[e]
        acc = y if acc is None else acc + y
    return acc


def _spatial_kernel(xs_ref, spwt_ref, wvt_ref, bv_ref, wot_ref, bo_ref,
                    ln2g_ref, ln2b_ref, ln3g_ref, ln3b_ref,
                    gw_ref, gb_ref, w1t_ref, b1_ref, w2t_ref, b2_ref,
                    p6_ref, out_ref):
    se = _dot(xs_ref[...], spwt_ref[...])
    a = _dot(se, wvt_ref[...]) + bv_ref[...]
    a = _dot(a, wot_ref[...]) + bo_ref[...]
    x1 = _ln(se + a, ln2g_ref[...], ln2b_ref[...])
    f = _moe_dense(x1, gw_ref[...], gb_ref[...], w1t_ref, b1_ref[...],
                   w2t_ref, b2_ref[...])
    sp = _ln(x1 + f, ln3g_ref[...], ln3b_ref[...])
    out_ref[...] = _dot_t(sp, p6_ref[...])


def _spatial_call(xs, p, proj_w):
    ap = p['attn']
    mp = p['moe']
    w1t = jnp.stack([e['W1'].T for e in mp['experts']])
    b1 = jnp.stack([e['b1'] for e in mp['experts']])
    w2t = jnp.stack([e['W2'].T for e in mp['experts']])
    b2 = jnp.stack([e['b2'] for e in mp['experts']])
    args = (xs, p['spatial_W_T'], ap['Wv'].T, ap['bv'][None], ap['Wo'].T,
            ap['bo'][None], p['ln2_g'][None], p['ln2_b'][None],
            p['ln3_g'][None], p['ln3_b'][None],
            mp['gate_W'], mp['gate_b'][None], w1t, b1, w2t, b2,
            proj_w[:, 5 * D:6 * D])
    return pl.pallas_call(
        _spatial_kernel,
        out_shape=jax.ShapeDtypeStruct((B * N, D), F32),
    )(*args)


NT = 34


def _embed_tile(trend_ref, season_ref, m6_ref, m2_ref, tod_idx_ref,
                dow_idx_ref, tod_tab_ref, dow_tab_ref, p4_ref, p5_ref,
                ut_ref, us_ref, u6_ref, u2_ref, cvec_ref, adp_ref, s_ref):
    ut = ut_ref[...].reshape(1, 1, D)
    us = us_ref[...].reshape(1, 1, D)
    h = trend_ref[0] * ut + season_ref[0] * us
    bl = m6_ref[0] * u6_ref[...] + m2_ref[0] * u2_ref[...] + cvec_ref[...]
    tp = _dot_t(tod_tab_ref[...], p4_ref[...])
    dp = _dot_t(dow_tab_ref[...], p5_ref[...])
    ti = tod_idx_ref[0]
    di = dow_idx_ref[0]
    oh_t = (jax.lax.broadcasted_iota(jnp.int32, (L, SLICE), 1) == ti
            ).astype(F32)
    oh_d = (jax.lax.broadcasted_iota(jnp.int32, (L, 7), 1) == di
            ).astype(F32)
    bl = bl + _dot(oh_t, tp) + _dot(oh_d, dp)
    h = h + bl[None, :, :]
    h = h + adp_ref[...]
    h = h + jnp.broadcast_to(s_ref[0], (NT, L, D))
    return h


def _mega_kernel(trend_ref, season_ref, m6_ref, m2_ref, tod_idx_ref,
                 dow_idx_ref, tod_tab_ref, dow_tab_ref, p4_ref, p5_ref,
                 ut_ref, us_ref, u6_ref, u2_ref, cvec_ref, adp_ref, s_ref,
                 wqt_ref, bq_ref, wkt_ref, bk_ref, wvt_ref, bv_ref,
                 wot_ref, bo_ref, ln2g_ref, ln2b_ref, ln3g_ref, ln3b_ref,
                 gw_ref, gb_ref, w1c_ref, b1c_ref, w2c_ref, b2s_ref,
                 ones_ref, ones96_ref, out_ref):
    h3 = _embed_tile(trend_ref, season_ref, m6_ref, m2_ref, tod_idx_ref,
                     dow_idx_ref, tod_tab_ref, dow_tab_ref, p4_ref, p5_ref,
                     ut_ref, us_ref, u6_ref, u2_ref, cvec_ref, adp_ref,
                     s_ref)
    x = h3.reshape(NT * L, D)
    ones_bf = ones_ref[...]
    ones96 = ones96_ref[...]
    for l in range(NLAYERS):
        x = _one_block(x, l, NT, wqt_ref, bq_ref, wkt_ref, bk_ref, wvt_ref,
                       bv_ref, wot_ref, bo_ref, ln2g_ref, ln2b_ref,
                       ln3g_ref, ln3b_ref, gw_ref, gb_ref, w1c_ref,
                       b1c_ref, w2c_ref, b2s_ref, ones_bf, ones96)
    out_ref[0] = x.reshape(NT, L, D)


def _mega_call(trend, season, m6, m2, tod_idx, dow_idx, params, folded,
               adaptive_p, s_proj):
    proj_w = params['proj_W']
    blocks = params['blocks']
    scale = 1.0 / (DH ** 0.5)
    wqt = jnp.stack([p['attn']['Wq'].T * scale for p in blocks]).astype(BF16)
    bq = jnp.stack([p['attn']['bq'][None] * scale for p in blocks])
    wkt = jnp.stack([p['attn']['Wk'].T for p in blocks]).astype(BF16)
    bk = jnp.stack([p['attn']['bk'][None] for p in blocks])
    wvt = jnp.stack([p['attn']['Wv'].T for p in blocks]).astype(BF16)
    bv = jnp.stack([p['attn']['bv'][None] for p in blocks])
    wot = jnp.stack([p['attn']['Wo'].T for p in blocks]).astype(BF16)
    bo = jnp.stack([p['attn']['bo'][None] for p in blocks])
    ln2g = jnp.stack([p['ln2_g'][None] for p in blocks])
    ln2b = jnp.stack([p['ln2_b'][None] for p in blocks])
    ln3g = jnp.stack([p['ln3_g'][None] for p in blocks])
    ln3b = jnp.stack([p['ln3_b'][None] for p in blocks])
    gw = jnp.stack([p['moe']['gate_W'] for p in blocks])
    gb = jnp.stack([p['moe']['gate_b'][:, None] for p in blocks])
    w1c = jnp.stack([
        jnp.concatenate([e['W1'].T for e in p['moe']['experts']], axis=1)
        for p in blocks]).astype(BF16)
    b1c = jnp.stack([
        jnp.concatenate([e['b1'] for e in p['moe']['experts']])[None]
        for p in blocks])
    w2c = jnp.stack([
        jnp.concatenate([e['W2'].T for e in p['moe']['experts']], axis=0)
        for p in blocks]).astype(BF16)
    b2s = jnp.stack([
        jnp.stack([e['b2'] for e in p['moe']['experts']]) for p in blocks])
    ones_bf = jnp.ones((D, D), BF16)
    ones96 = jnp.ones((L, DH), BF16)
    KK = NLAYERS
    full = lambda shape: pl.BlockSpec(
        shape, lambda b, j, _s=len(shape): tuple(0 for _ in range(_s)))
    bs = [
        pl.BlockSpec((1, NT, L, 1), lambda b, j: (b, j, 0, 0)),
        pl.BlockSpec((1, NT, L, 1), lambda b, j: (b, j, 0, 0)),
        pl.BlockSpec((1, L, 1), lambda b, j: (b, 0, 0)),
        pl.BlockSpec((1, L, 1), lambda b, j: (b, 0, 0)),
        pl.BlockSpec((1, L, 1), lambda b, j: (b, 0, 0)),
        pl.BlockSpec((1, L, 1), lambda b, j: (b, 0, 0)),
        full((SLICE, D)), full((7, D)), full((D, D)), full((D, D)),
        full((1, D)), full((1, D)), full((1, D)), full((1, D)), full((1, D)),
        pl.BlockSpec((NT, L, D), lambda b, j: (j, 0, 0)),
        pl.BlockSpec((1, NT, 1, D), lambda b, j: (b, j, 0, 0)),
        full((KK, D, D)), full((KK, 1, D)), full((KK, D, D)),
        full((KK, 1, D)), full((KK, D, D)), full((KK, 1, D)),
        full((KK, D, D)), full((KK, 1, D)),
        full((KK, 1, D)), full((KK, 1, D)), full((KK, 1, D)),
        full((KK, 1, D)),
        full((KK, E, D)), full((KK, E, 1)), full((KK, D, E * DFF)),
        full((KK, 1, E * DFF)), full((KK, E * DFF, D)), full((KK, E, D)),
        full((D, D)), full((L, DH)),
    ]
    ut, us, u6, u2, cvec = folded
    return pl.pallas_call(
        _mega_kernel,
        grid=(B, N // NT),
        in_specs=bs,
        out_specs=pl.BlockSpec((1, NT, L, D), lambda b, j: (b, j, 0, 0)),
        out_shape=jax.ShapeDtypeStruct((B, N, L, D), F32),
    )(trend[..., None], season[..., None], m6, m2, tod_idx, dow_idx,
      params['tod_table'], params['dow_table'],
      proj_w[:, 3 * D:4 * D], proj_w[:, 4 * D:5 * D],
      ut, us, u6, u2, cvec, adaptive_p, s_proj,
      wqt, bq, wkt, bk, wvt, bv, wot, bo, ln2g, ln2b, ln3g, ln3b,
      gw, gb, w1c, b1c, w2c, b2s, ones_bf, ones96)


G = 20


def _one_block(x, l, G, wqt_ref, bq_ref, wkt_ref, bk_ref, wvt_ref, bv_ref,
               wot_ref, bo_ref, ln2g_ref, ln2b_ref, ln3g_ref, ln3b_ref,
               gw_ref, gb_ref, w1c_ref, b1c_ref, w2c_ref, b2s_ref,
               ones_bf, ones96):
    xb = x.astype(BF16)
    q = _dotb(xb, wqt_ref[l]) + bq_ref[l]
    k = _dotb(xb, wkt_ref[l]) + bk_ref[l]
    v = _dotb(xb, wvt_ref[l]) + bv_ref[l]
    heads = []
    for h in range(H):
        qh = q[:, h * DH:(h + 1) * DH].reshape(G, L, DH)
        kh = k[:, h * DH:(h + 1) * DH].reshape(G, L, DH)
        vh = v[:, h * DH:(h + 1) * DH].reshape(G, L, DH)
        s = jax.lax.dot_general(qh.astype(BF16), kh.astype(BF16),
                                (((2,), (2,)), ((0,), (0,))),
                                preferred_element_type=F32)
        p = jnp.exp(s).astype(BF16)
        o = jax.lax.dot_general(p, vh.astype(BF16),
                                (((2,), (1,)), ((0,), (0,))),
                                preferred_element_type=F32)
        den = jax.lax.dot_general(p.reshape(G * L, L), ones96,
                                  (((1,), (0,)), ((), ())),
                                  preferred_element_type=F32)
        heads.append(o.reshape(G * L, DH) / den)
    a = jnp.concatenate(heads, axis=1)
    a = _dotb(a, wot_ref[l]) + bo_ref[l]
    x1 = _ln_mxu(x + a, ln2g_ref[l], ln2b_ref[l], ones_bf)
    logits_t = jax.lax.dot_general(gw_ref[l], x1, (((1,), (1,)), ((), ())),
                                   preferred_element_type=F32,
                                   precision=jax.lax.Precision.HIGHEST)
    logits_t = logits_t + gb_ref[l]
    w_t = _top2_weights_t(logits_t)
    w4 = jnp.transpose(w_t).astype(BF16)
    x1b = x1.astype(BF16)
    f = jax.lax.dot_general(w4, b2s_ref[l], (((1,), (0,)), ((), ())),
                            preferred_element_type=F32)
    for e in range(E):
        h1 = jnp.maximum(
            (_dotb(x1b, w1c_ref[l, :, e * DFF:(e + 1) * DFF])
             + b1c_ref[l, :, e * DFF:(e + 1) * DFF]).astype(BF16), 0)
        h1w = h1 * w4[:, e:e + 1]
        f = f + _dotb(h1w, w2c_ref[l, e * DFF:(e + 1) * DFF, :])
    return _ln_mxu(x1 + f, ln3g_ref[l], ln3b_ref[l], ones_bf)


def _mix_kernel(h_ref, wt_ref, b_ref, st_ref, mu_ref, out_ref):
    y = _dot(h_ref[0], wt_ref[...]) + b_ref[...]
    out_ref[0] = y * st_ref[0, 0, 0] + mu_ref[0, 0, 0]


def _mix_call(hflat, mix_w, mix_b, stdev, mean):
    bs = [
        pl.BlockSpec((1, N, L * D), lambda b: (b, 0, 0)),
        pl.BlockSpec((L * D, PRED), lambda b: (0, 0)),
        pl.BlockSpec((1, PRED), lambda b: (0, 0)),
        pl.BlockSpec((1, 1, 1), lambda b: (b, 0, 0)),
        pl.BlockSpec((1, 1, 1), lambda b: (b, 0, 0)),
    ]
    return pl.pallas_call(
        _mix_kernel,
        grid=(B,),
        in_specs=bs,
        out_specs=pl.BlockSpec((1, N, PRED), lambda b: (b, 0, 0)),
        out_shape=jax.ShapeDtypeStruct((B, N, PRED), F32),
    )(hflat, mix_w.T, mix_b[None], stdev, mean)


def _adp_kernel(a_ref, p_ref, out_ref):
    out_ref[...] = _dot_t(a_ref[...], p_ref[...])


def _adp_call(adaptive, p3):
    rows = N * L // 4
    return pl.pallas_call(
        _adp_kernel,
        grid=(4,),
        in_specs=[pl.BlockSpec((rows, D), lambda i: (i, 0)),
                  pl.BlockSpec((D, D), lambda i: (0, 0))],
        out_specs=pl.BlockSpec((rows, D), lambda i: (i, 0)),
        out_shape=jax.ShapeDtypeStruct((N * L, D), F32),
    )(adaptive, p3)


def kernel(x_enc, x_mark_enc, x_dec, x_mark_dec, params):
    x = x_enc
    mean = jnp.mean(x, axis=(1, 2), keepdims=True)
    xc = x - mean
    stdev = jnp.sqrt(jnp.mean(xc * xc, axis=(1, 2), keepdims=True) + EPS)
    xn = xc / stdev
    xr = jnp.swapaxes(xn, 1, 2)
    xp = jnp.pad(xr, ((0, 0), (0, 0), (1, 1)))
    trend = (xp[..., :-2] + xp[..., 1:-1] + xp[..., 2:]) / 3.0
    season = xr - trend

    m6 = x_mark_enc[:, :, 6]
    m2 = x_mark_enc[:, :, 2]
    tod_idx = (m6 * SLICE).astype(jnp.int32)[..., None]
    dow_idx = m2.astype(jnp.int32)[..., None]

    proj_w = params['proj_W']
    emb_w = params['emb_W']
    p01 = proj_w[:, :D] + proj_w[:, D:2 * D]
    ut = (proj_w[:, :D] @ emb_w[:, 0])[None]
    us = (proj_w[:, D:2 * D] @ emb_w[:, 0])[None]
    u6 = (p01 @ emb_w[:, 1])[None]
    u2 = (p01 @ emb_w[:, 2])[None]
    cvec = (p01 @ params['emb_b'] + params['proj_b'])[None]

    sp = dict(params['spatial_block'])
    sp['spatial_W_T'] = params['spatial_W'].T
    s_proj = _spatial_call(xr.reshape(B * N, L), sp, proj_w)

    adaptive_p = _adp_call(params['adaptive'], proj_w[:, 2 * D:3 * D])
    adaptive_p = adaptive_p.reshape(N, L, D)

    h = _mega_call(trend, season, m6[..., None], m2[..., None],
                   tod_idx, dow_idx, params, (ut, us, u6, u2, cvec),
                   adaptive_p, s_proj.reshape(B, N, 1, D))
    y = _mix_call(h.reshape(B, N, L * D), params['mix_W'], params['mix_b'],
                  stdev.reshape(B, 1, 1), mean.reshape(B, 1, 1))
    out = jnp.swapaxes(y, 1, 2)
    return out
